# Initial kernel scaffold; baseline (speedup 1.0000x reference)
import jax
import jax.numpy as jnp
from jax.experimental import pallas as pl


def kernel(x, edge_index, W):
    raise NotImplementedError("write your pallas kernel here")


# speedup vs baseline: 6.0728x; 6.0728x over previous
import jax
import jax.numpy as jnp
from jax import lax
from jax.experimental import pallas as pl
from jax.experimental.pallas import tpu as pltpu
from jax.experimental.pallas import tpu_sc as plsc

N_NODES = 10000
N_EDGES = 320000
FEATS = 128

NC = 2
NS = 16
NW = NC * NS

EPT = N_EDGES // NW
EB = 80
NBATCH = EPT // EB
NPT = 624
NTAIL = N_NODES - NS * NPT
HR = 80


def _sc_body(x_hbm, src_hbm, dst_hbm, z128_hbm, iota_hbm,
             psum_hbm, pdeg_hbm,
             sidx, didx, rows, hist, iov, acc, dacc, sem):
    c = lax.axis_index("c")
    s = lax.axis_index("s")
    wid = c * NS + s

    rbase = pl.multiple_of(s * NPT, 8)
    tail = NS * NPT
    pltpu.sync_copy(z128_hbm.at[pl.ds(rbase, NPT)],
                    acc.at[pl.ds(rbase, NPT)])

    @pl.when(s == NS - 1)
    def _zero_tail():
        pltpu.sync_copy(z128_hbm.at[pl.ds(tail, NTAIL)],
                        acc.at[pl.ds(tail, NTAIL)])

    @pl.when(s == 0)
    def _zero_deg():
        pltpu.sync_copy(z128_hbm.at[pl.ds(0, HR)], dacc)

    pltpu.sync_copy(z128_hbm.at[pl.ds(0, HR)], hist)
    pltpu.sync_copy(iota_hbm, iov)
    plsc.subcore_barrier()

    ebase = wid * EPT

    def batch(j, carry):
        off = pl.multiple_of(ebase + j * EB, 8)
        pltpu.sync_copy(src_hbm.at[pl.ds(off, EB)], sidx)
        pltpu.sync_copy(dst_hbm.at[pl.ds(off, EB)], didx)
        pltpu.async_copy(x_hbm.at[sidx], rows, sem).wait()
        pltpu.sync_copy(rows, acc.at[didx], add=True)
        for v in range(EB // 16):
            d = didx[pl.ds(v * 16, 16)]
            row = lax.shift_right_logical(d, 7)
            col = lax.bitwise_and(d, 127)
            cnt, last = plsc.scan_count(d)
            plsc.addupdate_scatter(hist, [row, col],
                                   cnt.astype(jnp.float32), mask=last)
        return carry

    lax.fori_loop(0, NBATCH, batch, 0)
    pltpu.sync_copy(hist, dacc.at[iov], add=True)
    plsc.subcore_barrier()

    pltpu.sync_copy(acc.at[pl.ds(rbase, NPT)],
                    psum_hbm.at[c, pl.ds(rbase, NPT)])

    @pl.when(s == NS - 1)
    def _write_tail():
        pltpu.sync_copy(acc.at[pl.ds(tail, NTAIL)],
                        psum_hbm.at[c, pl.ds(tail, NTAIL)])

    @pl.when(s == 0)
    def _write_deg():
        pltpu.sync_copy(dacc, pdeg_hbm.at[c])


@jax.jit
def _sc_aggregate(x, src, dst, z128, iota):
    mesh = plsc.VectorSubcoreMesh(core_axis_name="c", subcore_axis_name="s")
    return pl.kernel(
        _sc_body,
        out_type=(
            jax.ShapeDtypeStruct((NC, N_NODES, FEATS), jnp.float32),
            jax.ShapeDtypeStruct((NC, HR, FEATS), jnp.float32),
        ),
        mesh=mesh,
        compiler_params=pltpu.CompilerParams(needs_layout_passes=False),
        scratch_types=[
            pltpu.VMEM((EB,), jnp.int32),
            pltpu.VMEM((EB,), jnp.int32),
            pltpu.VMEM((EB, FEATS), jnp.float32),
            pltpu.VMEM((HR, FEATS), jnp.float32),
            pltpu.VMEM((HR,), jnp.int32),
            pltpu.VMEM_SHARED((N_NODES, FEATS), jnp.float32),
            pltpu.VMEM_SHARED((HR, FEATS), jnp.float32),
            pltpu.SemaphoreType.DMA,
        ],
    )(x, src, dst, z128, iota)


def _tc_body(p0_ref, p1_ref, d0_ref, d1_ref, w_ref, out_ref):
    ssum = p0_ref[...] + p1_ref[...]
    deg = d0_ref[...] + d1_ref[...]
    deg = jnp.maximum(deg, 1.0)
    agg = ssum / deg
    out_ref[...] = lax.dot_general(
        agg, w_ref[...], (((1,), (1,)), ((), ())),
        preferred_element_type=jnp.float32)


@jax.jit
def _tc_finish(p0, p1, d0, d1, W):
    BN = 2000
    grid = (N_NODES // BN,)
    return pl.pallas_call(
        _tc_body,
        grid=grid,
        in_specs=[
            pl.BlockSpec((BN, FEATS), lambda i: (i, 0)),
            pl.BlockSpec((BN, FEATS), lambda i: (i, 0)),
            pl.BlockSpec((BN, 1), lambda i: (i, 0)),
            pl.BlockSpec((BN, 1), lambda i: (i, 0)),
            pl.BlockSpec((FEATS, FEATS), lambda i: (0, 0)),
        ],
        out_specs=pl.BlockSpec((BN, FEATS), lambda i: (i, 0)),
        out_shape=jax.ShapeDtypeStruct((N_NODES, FEATS), jnp.float32),
    )(p0, p1, d0, d1, W)


def kernel(x, edge_index, W):
    src = edge_index[0].astype(jnp.int32)
    dst = edge_index[1].astype(jnp.int32)
    z128 = jnp.zeros((N_NODES, FEATS), jnp.float32)
    iota = jnp.arange(HR, dtype=jnp.int32)
    psum, pdeg = _sc_aggregate(x, src, dst, z128, iota)
    deg = pdeg.reshape(NC, HR * FEATS)[:, :N_NODES]
    return _tc_finish(psum[0], psum[1], deg[0][:, None], deg[1][:, None], W)


# speedup vs baseline: 11.0615x; 1.8215x over previous
import jax
import jax.numpy as jnp
from jax import lax
from jax.experimental import pallas as pl
from jax.experimental.pallas import tpu as pltpu
from jax.experimental.pallas import tpu_sc as plsc

N_NODES = 10000
N_EDGES = 320000
FEATS = 128

NC = 2
NS = 16
NW = NC * NS

EPT = N_EDGES // NW
EB = 80
NBATCH = EPT // EB
NPT = 624
NTAIL = N_NODES - NS * NPT
HR = 80


def _sc_body(x_hbm, src_hbm, dst_hbm, z128_hbm, iota_hbm,
             psum_hbm, pdeg_hbm,
             sidx_all, didx0, didx1, rows0, rows1, hist, iov, acc, dacc,
             sem0, sem1):
    c = lax.axis_index("c")
    s = lax.axis_index("s")
    wid = c * NS + s
    didx = (didx0, didx1)
    rows = (rows0, rows1)
    sem = (sem0, sem1)

    rbase = pl.multiple_of(s * NPT, 8)
    tail = NS * NPT
    pltpu.sync_copy(z128_hbm.at[pl.ds(rbase, NPT)],
                    acc.at[pl.ds(rbase, NPT)])

    @pl.when(s == NS - 1)
    def _zero_tail():
        pltpu.sync_copy(z128_hbm.at[pl.ds(tail, NTAIL)],
                        acc.at[pl.ds(tail, NTAIL)])

    @pl.when(s == 0)
    def _zero_deg():
        pltpu.sync_copy(z128_hbm.at[pl.ds(0, HR)], dacc)

    pltpu.sync_copy(z128_hbm.at[pl.ds(0, HR)], hist)
    pltpu.sync_copy(iota_hbm, iov)

    ebase = pl.multiple_of(wid * EPT, 8)
    pltpu.sync_copy(src_hbm.at[pl.ds(ebase, EPT)], sidx_all)
    plsc.subcore_barrier()

    def start(j, b):
        off = pl.multiple_of(ebase + j * EB, 8)
        pltpu.sync_copy(dst_hbm.at[pl.ds(off, EB)], didx[b])
        loc = pl.multiple_of(j * EB, 8)
        pltpu.async_copy(x_hbm.at[sidx_all.at[pl.ds(loc, EB)]],
                         rows[b], sem[b])

    def consume(j, b):
        loc = pl.multiple_of(j * EB, 8)
        pltpu.make_async_copy(x_hbm.at[sidx_all.at[pl.ds(loc, EB)]],
                              rows[b], sem[b]).wait()
        pltpu.sync_copy(rows[b], acc.at[didx[b]], add=True)
        for v in range(EB // 16):
            d = didx[b][pl.ds(v * 16, 16)]
            row = lax.shift_right_logical(d, 7)
            col = lax.bitwise_and(d, 127)
            cnt, last = plsc.scan_count(d)
            plsc.addupdate_scatter(hist, [row, col],
                                   cnt.astype(jnp.float32), mask=last)

    start(0, 0)
    start(1, 1)

    def pair(i, carry):
        j = i * 2
        consume(j, 0)

        @pl.when(j + 2 < NBATCH)
        def _pf0():
            start(j + 2, 0)

        consume(j + 1, 1)

        @pl.when(j + 3 < NBATCH)
        def _pf1():
            start(j + 3, 1)

        return carry

    lax.fori_loop(0, NBATCH // 2, pair, 0)
    if NBATCH % 2:
        consume(NBATCH - 1, 0)
    pltpu.sync_copy(hist, dacc.at[iov], add=True)
    plsc.subcore_barrier()

    pltpu.sync_copy(acc.at[pl.ds(rbase, NPT)],
                    psum_hbm.at[c, pl.ds(rbase, NPT)])

    @pl.when(s == NS - 1)
    def _write_tail():
        pltpu.sync_copy(acc.at[pl.ds(tail, NTAIL)],
                        psum_hbm.at[c, pl.ds(tail, NTAIL)])

    @pl.when(s == 0)
    def _write_deg():
        pltpu.sync_copy(dacc, pdeg_hbm.at[c])


@jax.jit
def _sc_aggregate(x, src, dst, z128, iota):
    mesh = plsc.VectorSubcoreMesh(core_axis_name="c", subcore_axis_name="s")
    return pl.kernel(
        _sc_body,
        out_type=(
            jax.ShapeDtypeStruct((NC, N_NODES, FEATS), jnp.float32),
            jax.ShapeDtypeStruct((NC, HR, FEATS), jnp.float32),
        ),
        mesh=mesh,
        compiler_params=pltpu.CompilerParams(needs_layout_passes=False),
        scratch_types=[
            pltpu.VMEM((EPT,), jnp.int32),
            pltpu.VMEM((EB,), jnp.int32),
            pltpu.VMEM((EB,), jnp.int32),
            pltpu.VMEM((EB, FEATS), jnp.float32),
            pltpu.VMEM((EB, FEATS), jnp.float32),
            pltpu.VMEM((HR, FEATS), jnp.float32),
            pltpu.VMEM((HR,), jnp.int32),
            pltpu.VMEM_SHARED((N_NODES, FEATS), jnp.float32),
            pltpu.VMEM_SHARED((HR, FEATS), jnp.float32),
            pltpu.SemaphoreType.DMA,
            pltpu.SemaphoreType.DMA,
        ],
    )(x, src, dst, z128, iota)


def _tc_body(p0_ref, p1_ref, d0_ref, d1_ref, w_ref, out_ref):
    ssum = p0_ref[...] + p1_ref[...]
    deg = d0_ref[...] + d1_ref[...]
    deg = jnp.maximum(deg, 1.0)
    agg = ssum / deg
    out_ref[...] = lax.dot_general(
        agg, w_ref[...], (((1,), (1,)), ((), ())),
        preferred_element_type=jnp.float32)


@jax.jit
def _tc_finish(p0, p1, d0, d1, W):
    BN = 2000
    grid = (N_NODES // BN,)
    return pl.pallas_call(
        _tc_body,
        grid=grid,
        in_specs=[
            pl.BlockSpec((BN, FEATS), lambda i: (i, 0)),
            pl.BlockSpec((BN, FEATS), lambda i: (i, 0)),
            pl.BlockSpec((BN, 1), lambda i: (i, 0)),
            pl.BlockSpec((BN, 1), lambda i: (i, 0)),
            pl.BlockSpec((FEATS, FEATS), lambda i: (0, 0)),
        ],
        out_specs=pl.BlockSpec((BN, FEATS), lambda i: (i, 0)),
        out_shape=jax.ShapeDtypeStruct((N_NODES, FEATS), jnp.float32),
    )(p0, p1, d0, d1, W)


def kernel(x, edge_index, W):
    src = edge_index[0].astype(jnp.int32)
    dst = edge_index[1].astype(jnp.int32)
    z128 = jnp.zeros((N_NODES, FEATS), jnp.float32)
    iota = jnp.arange(HR, dtype=jnp.int32)
    psum, pdeg = _sc_aggregate(x, src, dst, z128, iota)
    deg = pdeg.reshape(NC, HR * FEATS)[:, :N_NODES]
    return _tc_finish(psum[0], psum[1], deg[0][:, None], deg[1][:, None], W)
